# baseline (speedup 1.0000x reference)
import functools

import jax
import jax.numpy as jnp
from jax import lax
from jax.experimental import pallas as pl
from jax.experimental.pallas import tpu as pltpu
from jax.experimental.pallas import tpu_sc as plsc

_MAX_SEQ_LENGTH = 2048


def _sc_pad_body(x_hbm, z_hbm, o_hbm, zseed, copy_sem, zero_sem, seed_sem):
    b, s, f = x_hbm.shape
    pad = _MAX_SEQ_LENGTH - s
    info = plsc.get_sparse_core_info()
    nc, ns = info.num_cores, info.num_subcores
    nw = nc * ns
    rows_c = s // (nw // b)
    rows_z = pad // (nw // b)
    wpr = nw // b

    cid = lax.axis_index("c")
    sid = lax.axis_index("s")
    w = sid * nc + cid

    @pl.when(sid == 0)
    def _seed():
        pltpu.async_copy(z_hbm, zseed, seed_sem).wait()

    plsc.subcore_barrier()

    i = w // wpr
    q = w % wpr

    cp = pltpu.make_async_copy(
        x_hbm.at[pl.ds(i, 1), pl.ds(q * rows_c, rows_c)],
        o_hbm.at[pl.ds(i, 1), pl.ds(q * rows_c, rows_c)],
        copy_sem,
    )
    cp.start()
    zp = pltpu.make_async_copy(
        zseed,
        o_hbm.at[pl.ds(i, 1), pl.ds(s + q * rows_z, rows_z)],
        zero_sem,
    )
    zp.start()
    cp.wait()
    zp.wait()


def kernel(x):
    b, s, f = x.shape
    out_s = _MAX_SEQ_LENGTH
    pad = out_s - s
    info = plsc.get_sparse_core_info()
    nw = info.num_cores * info.num_subcores
    rows_z = pad // (nw // b)

    zeros_seed = jnp.zeros((1, rows_z, f), x.dtype)

    sc_kernel = pl.kernel(
        _sc_pad_body,
        out_type=jax.ShapeDtypeStruct((b, out_s, f), x.dtype),
        mesh=plsc.VectorSubcoreMesh(core_axis_name="c", subcore_axis_name="s"),
        scratch_types=[
            pltpu.VMEM_SHARED((1, rows_z, f), x.dtype),
            pltpu.SemaphoreType.DMA,
            pltpu.SemaphoreType.DMA,
            pltpu.SemaphoreType.DMA,
        ],
    )
    return sc_kernel(x, zeros_seed)


# speedup vs baseline: 18.7790x; 18.7790x over previous
import jax
import jax.numpy as jnp
from jax import lax
from jax.experimental import pallas as pl
from jax.experimental.pallas import tpu as pltpu
from jax.experimental.pallas import tpu_sc as plsc

_MAX_SEQ_LENGTH = 2048
_CHUNK_ROWS = 32


def _sc_pad_body(
    x_hbm, z_hbm, o_hbm, zseed, bufs, in_sem, out_sem, zero_sem, seed_sem
):
    b, s, f = x_hbm.shape
    pad = _MAX_SEQ_LENGTH - s
    info = plsc.get_sparse_core_info()
    nc, ns = info.num_cores, info.num_subcores
    nw = nc * ns
    wpr = nw // b
    rows_c = s // wpr
    rows_z = pad // wpr
    cr = _CHUNK_ROWS
    nchunks = rows_c // cr

    cid = lax.axis_index("c")
    sid = lax.axis_index("s")
    w = sid * nc + cid

    @pl.when(sid == 0)
    def _seed():
        pltpu.async_copy(z_hbm, zseed, seed_sem).wait()

    plsc.subcore_barrier()

    i = w // wpr
    q = w % wpr
    base = q * rows_c

    zp = pltpu.make_async_copy(
        zseed,
        o_hbm.at[pl.ds(i, 1), pl.ds(s + q * rows_z, rows_z)],
        zero_sem,
    )
    zp.start()

    def cin(k):
        return pltpu.make_async_copy(
            x_hbm.at[pl.ds(i, 1), pl.ds(base + k * cr, cr)],
            bufs.at[k % 2],
            in_sem,
        )

    def cout(k):
        return pltpu.make_async_copy(
            bufs.at[k % 2],
            o_hbm.at[pl.ds(i, 1), pl.ds(base + k * cr, cr)],
            out_sem,
        )

    cin(0).start()
    for k in range(nchunks):
        if k + 1 < nchunks:
            if k >= 1:
                cout(k - 1).wait()
            cin(k + 1).start()
        cin(k).wait()
        cout(k).start()

    for k in range(max(0, nchunks - 2), nchunks):
        cout(k).wait()
    zp.wait()


def kernel(x):
    b, s, f = x.shape
    out_s = _MAX_SEQ_LENGTH
    pad = out_s - s
    info = plsc.get_sparse_core_info()
    nw = info.num_cores * info.num_subcores
    rows_z = pad // (nw // b)

    zeros_seed = jnp.zeros((1, rows_z, f), x.dtype)

    sc_kernel = pl.kernel(
        _sc_pad_body,
        out_type=jax.ShapeDtypeStruct((b, out_s, f), x.dtype),
        mesh=plsc.VectorSubcoreMesh(core_axis_name="c", subcore_axis_name="s"),
        scratch_types=[
            pltpu.VMEM_SHARED((1, rows_z, f), x.dtype),
            pltpu.VMEM((2, 1, _CHUNK_ROWS, f), x.dtype),
            pltpu.SemaphoreType.DMA,
            pltpu.SemaphoreType.DMA,
            pltpu.SemaphoreType.DMA,
            pltpu.SemaphoreType.DMA,
        ],
    )
    return sc_kernel(x, zeros_seed)


# speedup vs baseline: 34.1218x; 1.8170x over previous
import jax
import jax.numpy as jnp
from jax.experimental import pallas as pl
from jax.experimental.pallas import tpu as pltpu

_MAX_SEQ_LENGTH = 2048
_CHUNK_S = 512
_ZERO_S = 128


def _pad_dma_body(x_hbm, o_hbm, bufs, zeros_vmem, in_sem, out_sem, zero_sem):
    b, s, f = x_hbm.shape
    pad = _MAX_SEQ_LENGTH - s
    cs = _CHUNK_S
    cpr = s // cs
    n = b * cpr
    zs = _ZERO_S
    zpr = pad // zs
    n_zero = b * zpr

    def in_copy(t):
        i, j = divmod(t, cpr)
        return pltpu.make_async_copy(
            x_hbm.at[pl.ds(i, 1), pl.ds(j * cs, cs)], bufs.at[t], in_sem
        )

    def out_copy(t):
        i, j = divmod(t, cpr)
        return pltpu.make_async_copy(
            bufs.at[t], o_hbm.at[pl.ds(i, 1), pl.ds(j * cs, cs)], out_sem
        )

    def zero_copy(k):
        i, j = divmod(k, zpr)
        return pltpu.make_async_copy(
            zeros_vmem, o_hbm.at[pl.ds(i, 1), pl.ds(s + j * zs, zs)], zero_sem
        )

    in_copy(0).start()
    in_copy(1).start()

    zeros_vmem[...] = jnp.zeros_like(zeros_vmem)
    n_head = 4
    for k in range(n_head):
        zero_copy(k).start()

    for t in range(2, n):
        in_copy(t).start()

    zk = n_head
    per_iter = (n_zero - n_head + n - 1) // n
    for t in range(n):
        in_copy(t).wait()
        out_copy(t).start()
        for _ in range(per_iter):
            if zk < n_zero:
                zero_copy(zk).start()
                zk += 1
    while zk < n_zero:
        zero_copy(zk).start()
        zk += 1

    for t in range(n):
        out_copy(t).wait()
    for k in range(n_zero):
        zero_copy(k).wait()


def kernel(x):
    b, s, f = x.shape
    out_s = _MAX_SEQ_LENGTH
    cs = _CHUNK_S
    n = (s // cs) * b

    return pl.pallas_call(
        _pad_dma_body,
        in_specs=[pl.BlockSpec(memory_space=pltpu.MemorySpace.HBM)],
        out_specs=pl.BlockSpec(memory_space=pltpu.MemorySpace.HBM),
        out_shape=jax.ShapeDtypeStruct((b, out_s, f), x.dtype),
        scratch_shapes=[
            pltpu.VMEM((n, 1, cs, f), x.dtype),
            pltpu.VMEM((1, _ZERO_S, f), x.dtype),
            pltpu.SemaphoreType.DMA,
            pltpu.SemaphoreType.DMA,
            pltpu.SemaphoreType.DMA,
        ],
    )(x)
